# baseline (speedup 1.0000x reference)
import jax
import jax.numpy as jnp
import numpy as np
from jax.experimental import pallas as pl
from jax.experimental.pallas import tpu as pltpu

STRIDES = (8, 16, 32, 64, 128)
FEATURE_SHAPES = ((128, 128), (64, 64), (32, 32), (16, 16), (8, 8))
POS_SCALE = 0.2
IGNORE_SCALE = 0.5
NUM_CLASSES = 80
TILE = 64
NUM_LOC = sum(h * w for h, w in FEATURE_SHAPES)
NUM_TILES = NUM_LOC // TILE
GROUP = 16
NUM_GROUPS = -(-NUM_TILES // GROUP)


def _build_meta():
    cols = []
    for stride, (fh, fw) in zip(STRIDES, FEATURE_SHAPES):
        ys, xs = np.meshgrid(np.arange(fh), np.arange(fw), indexing="ij")
        xs = xs.reshape(-1).astype(np.float32)
        ys = ys.reshape(-1).astype(np.float32)
        cols.append(
            np.stack([xs, ys, (xs + 0.5) * stride, (ys + 0.5) * stride],
                     axis=1))
    meta = np.concatenate(cols, axis=0)
    pad = NUM_GROUPS * GROUP * TILE - NUM_LOC
    meta = np.concatenate([meta, np.full((pad, 4), -1.0, np.float32)],
                          axis=0)
    return meta.reshape(NUM_GROUPS, GROUP, TILE, 4)


def _build_tile_levels():
    lv = []
    for lid, (fh, fw) in enumerate(FEATURE_SHAPES):
        lv += [lid] * ((fh * fw) // TILE)
    lv += [len(STRIDES) - 1] * (NUM_GROUPS * GROUP - NUM_TILES)
    return np.asarray(lv, dtype=np.int32)


_META = _build_meta()
_TILE_LVL = _build_tile_levels()


def _fsaf_group(lvl_sm, box_rows_ref, box_cols_ref, meta_ref,
                cls_ref, aux_ref, npos_ref, bnd):
    c = pl.program_id(1)
    nb = box_cols_ref.shape[1]

    @pl.when(c == 0)
    def _prologue():
        rows = box_rows_ref[0]
        x1 = rows[0:1, :]
        y1 = rows[1:2, :]
        x2 = rows[2:3, :]
        y2 = rows[3:4, :]
        blv = rows[5:6, :]
        cols = box_cols_ref[0]
        a_col = ((cols[:, 2:3] - cols[:, 0:1])
                 * (cols[:, 3:4] - cols[:, 1:2]))
        a_row = (x2 - x1) * (y2 - y1)
        li = jax.lax.broadcasted_iota(jnp.int32, (nb, nb), 1)
        si = jax.lax.broadcasted_iota(jnp.int32, (nb, nb), 0)
        before = (a_col < a_row) | ((a_col == a_row) & (si < li))
        bnd[40:41, :] = jnp.sum(before.astype(jnp.float32), axis=0,
                                keepdims=True)
        for lid, (stride, (fh, fw)) in enumerate(zip(STRIDES,
                                                     FEATURE_SHAPES)):
            inv = 1.0 / stride
            px1 = x1 * inv
            py1 = y1 * inv
            px2 = x2 * inv
            py2 = y2 * inv
            cx = (px1 + px2) * 0.5
            cy = (py1 + py2) * 0.5
            dw = px2 - px1
            dh = py2 - py1
            at_l = blv == float(lid)
            for scale, base in ((POS_SCALE, 0), (IGNORE_SCALE, 20)):
                hw = dw * scale * 0.5
                hh = dh * scale * 0.5
                bx1 = jnp.clip(jnp.floor(cx - hw), 0.0, fw - 1.0)
                by1 = jnp.clip(jnp.floor(cy - hh), 0.0, fh - 1.0)
                bx2 = jnp.clip(jnp.ceil(cx + hw), 1.0, float(fw))
                by2 = jnp.clip(jnp.ceil(cy + hh), 1.0, float(fh))
                bx2 = jnp.maximum(bx2, bx1 + 1.0)
                by2 = jnp.maximum(by2, by1 + 1.0)
                r = base + 4 * lid
                bnd[r:r + 1, :] = jnp.where(at_l, bx1, 1e9)
                bnd[r + 1:r + 2, :] = by1
                bnd[r + 2:r + 3, :] = bx2
                bnd[r + 3:r + 4, :] = by2

    box_cols = box_cols_ref[0]
    rank = bnd[40:41, :]
    npos_parts = []
    for k in range(GROUP):
        meta = meta_ref[0, k]
        xc = meta[:, 0:1]
        yc = meta[:, 1:2]
        sx = meta[:, 2:3]
        sy = meta[:, 3:4]

        lvl4 = lvl_sm[c * GROUP + k] * 4
        p = bnd[pl.ds(lvl4, 4), :]
        q = bnd[pl.ds(lvl4 + 20, 4), :]

        in_pos = ((xc >= p[0:1, :]) & (xc < p[2:3, :])
                  & (yc >= p[1:2, :]) & (yc < p[3:4, :]))
        in_ign = ((xc >= q[0:1, :]) & (xc < q[2:3, :])
                  & (yc >= q[1:2, :]) & (yc < q[3:4, :]))

        key = jnp.where(in_ign, jnp.where(in_pos, rank, 1000.0), 1e9)
        sel = jnp.min(key, axis=1, keepdims=True)
        onehot = (rank == sel).astype(jnp.float32)
        sel_pos = (sel < 1000.0).astype(jnp.float32)

        fields = jnp.dot(onehot, box_cols,
                         preferred_element_type=jnp.float32,
                         precision=jax.lax.Precision.HIGHEST)
        sx1 = fields[:, 0:1]
        sy1 = fields[:, 1:2]
        sx2 = fields[:, 2:3]
        sy2 = fields[:, 3:4]
        slab = fields[:, 4:5]

        lch = (sx - sx1) / 4.0 * sel_pos
        tch = (sy - sy1) / 4.0 * sel_pos
        rch = (sx2 - sx) / 4.0 * sel_pos
        bch = (sy2 - sy) / 4.0 * sel_pos

        cls_iota = jax.lax.broadcasted_iota(
            jnp.int32, (TILE, NUM_CLASSES), 1).astype(jnp.float32)
        cls_ref[0, k * TILE:(k + 1) * TILE, :] = (
            (cls_iota == slab).astype(jnp.float32) * sel_pos)

        cls_m = 1.0 - (sel == 1000.0).astype(jnp.float32)
        zero = jnp.zeros_like(sel_pos)
        aux_ref[0, k * TILE:(k + 1) * TILE, :] = jnp.concatenate(
            [lch, tch, rch, bch, cls_m, sel_pos, zero, zero], axis=1)

        npos_parts.append(sel_pos)

    part = jnp.sum(sum(npos_parts)).reshape(1, 1, 1)

    @pl.when(c == 0)
    def _init():
        npos_ref[...] = part

    @pl.when(c != 0)
    def _acc():
        npos_ref[...] += part


def kernel(gt_box_levels, gt_boxes, feature_shapes):
    del feature_shapes
    batch, nb = gt_box_levels.shape
    rows = jnp.concatenate(
        [jnp.transpose(gt_boxes, (0, 2, 1)),
         gt_box_levels[:, None, :].astype(jnp.float32)],
        axis=1,
    )
    meta = jnp.asarray(_META)
    tile_lvl = jnp.asarray(_TILE_LVL)

    blk = GROUP * TILE
    grid_spec = pltpu.PrefetchScalarGridSpec(
        num_scalar_prefetch=1,
        grid=(batch, NUM_GROUPS),
        in_specs=[
            pl.BlockSpec((1, 6, nb), lambda b, c, s: (b, 0, 0)),
            pl.BlockSpec((1, nb, 5), lambda b, c, s: (b, 0, 0)),
            pl.BlockSpec((1, GROUP, TILE, 4),
                         lambda b, c, s: (c, 0, 0, 0)),
        ],
        out_specs=[
            pl.BlockSpec((1, blk, NUM_CLASSES), lambda b, c, s: (b, c, 0)),
            pl.BlockSpec((1, blk, 8), lambda b, c, s: (b, c, 0)),
            pl.BlockSpec((1, 1, 1), lambda b, c, s: (b, 0, 0)),
        ],
        scratch_shapes=[pltpu.VMEM((48, nb), jnp.float32)],
    )

    cls_t, aux, num_pos = pl.pallas_call(
        _fsaf_group,
        grid_spec=grid_spec,
        out_shape=[
            jax.ShapeDtypeStruct((batch, NUM_LOC, NUM_CLASSES), jnp.float32),
            jax.ShapeDtypeStruct((batch, NUM_LOC, 8), jnp.float32),
            jax.ShapeDtypeStruct((batch, 1, 1), jnp.float32),
        ],
    )(tile_lvl, rows, gt_boxes, meta)

    return (cls_t,
            aux[..., 4] != 0.0,
            num_pos[:, 0, 0],
            aux[..., 0:4],
            aux[..., 5] != 0.0)


# speedup vs baseline: 1.3342x; 1.3342x over previous
import jax
import jax.numpy as jnp
import numpy as np
from jax.experimental import pallas as pl
from jax.experimental.pallas import tpu as pltpu

STRIDES = (8, 16, 32, 64, 128)
FEATURE_SHAPES = ((128, 128), (64, 64), (32, 32), (16, 16), (8, 8))
POS_SCALE = 0.2
IGNORE_SCALE = 0.5
NUM_CLASSES = 80
TILE = 64
NUM_LOC = sum(h * w for h, w in FEATURE_SHAPES)
NUM_TILES = NUM_LOC // TILE
GROUP = 8
NUM_GROUPS = -(-NUM_TILES // GROUP)


def _build_meta():
    cols = []
    for stride, (fh, fw) in zip(STRIDES, FEATURE_SHAPES):
        ys, xs = np.meshgrid(np.arange(fh), np.arange(fw), indexing="ij")
        xs = xs.reshape(-1).astype(np.float32)
        ys = ys.reshape(-1).astype(np.float32)
        cols.append(
            np.stack([xs, ys, (xs + 0.5) * stride, (ys + 0.5) * stride],
                     axis=1))
    meta = np.concatenate(cols, axis=0)
    pad = NUM_GROUPS * GROUP * TILE - NUM_LOC
    meta = np.concatenate([meta, np.full((pad, 4), -1.0, np.float32)],
                          axis=0)
    return meta.reshape(NUM_GROUPS, GROUP, TILE, 4)


def _build_tile_levels():
    lv = []
    for lid, (fh, fw) in enumerate(FEATURE_SHAPES):
        lv += [lid] * ((fh * fw) // TILE)
    lv += [len(STRIDES) - 1] * (NUM_GROUPS * GROUP - NUM_TILES)
    return np.asarray(lv, dtype=np.int32)


_META = _build_meta()
_TILE_LVL = _build_tile_levels()


def _fsaf_group(lvl_sm, box_rows_ref, box_cols_ref, meta_ref,
                cls_ref, aux_ref, npos_ref, bnd, lmat):
    c = pl.program_id(1)
    nb = box_cols_ref.shape[1]

    @pl.when(c == 0)
    def _prologue():
        rows = box_rows_ref[0]
        x1 = rows[0:1, :]
        y1 = rows[1:2, :]
        x2 = rows[2:3, :]
        y2 = rows[3:4, :]
        blv = rows[5:6, :]
        cols = box_cols_ref[0]
        a_col = ((cols[:, 2:3] - cols[:, 0:1])
                 * (cols[:, 3:4] - cols[:, 1:2]))
        a_row = (x2 - x1) * (y2 - y1)
        li = jax.lax.broadcasted_iota(jnp.int32, (nb, nb), 1)
        si = jax.lax.broadcasted_iota(jnp.int32, (nb, nb), 0)
        before = (a_col < a_row) | ((a_col == a_row) & (si < li))
        bnd[40:41, :] = jnp.sum(before.astype(jnp.float32), axis=0,
                                keepdims=True)
        cls_iota = jax.lax.broadcasted_iota(jnp.int32, (nb, NUM_CLASSES),
                                            1).astype(jnp.float32)
        lmat[...] = (cls_iota == cols[:, 4:5]).astype(jnp.float32)
        for lid, (stride, (fh, fw)) in enumerate(zip(STRIDES,
                                                     FEATURE_SHAPES)):
            inv = 1.0 / stride
            px1 = x1 * inv
            py1 = y1 * inv
            px2 = x2 * inv
            py2 = y2 * inv
            cx = (px1 + px2) * 0.5
            cy = (py1 + py2) * 0.5
            dw = px2 - px1
            dh = py2 - py1
            at_l = blv == float(lid)
            for scale, base in ((POS_SCALE, 0), (IGNORE_SCALE, 20)):
                hw = dw * scale * 0.5
                hh = dh * scale * 0.5
                bx1 = jnp.clip(jnp.floor(cx - hw), 0.0, fw - 1.0)
                by1 = jnp.clip(jnp.floor(cy - hh), 0.0, fh - 1.0)
                bx2 = jnp.clip(jnp.ceil(cx + hw), 1.0, float(fw))
                by2 = jnp.clip(jnp.ceil(cy + hh), 1.0, float(fh))
                bx2 = jnp.maximum(bx2, bx1 + 1.0)
                by2 = jnp.maximum(by2, by1 + 1.0)
                r = base + 4 * lid
                bnd[r:r + 1, :] = jnp.where(at_l, bx1, 1e9)
                bnd[r + 1:r + 2, :] = by1
                bnd[r + 2:r + 3, :] = bx2
                bnd[r + 3:r + 4, :] = by2

    box_cols = box_cols_ref[0]
    rank = bnd[40:41, :]
    npos_parts = []
    for k in range(GROUP):
        meta = meta_ref[0, k]
        xc = meta[:, 0:1]
        yc = meta[:, 1:2]
        sx = meta[:, 2:3]
        sy = meta[:, 3:4]

        lvl4 = lvl_sm[c * GROUP + k] * 4
        p = bnd[pl.ds(lvl4, 4), :]
        q = bnd[pl.ds(lvl4 + 20, 4), :]

        in_pos = ((xc >= p[0:1, :]) & (xc < p[2:3, :])
                  & (yc >= p[1:2, :]) & (yc < p[3:4, :]))
        in_ign = ((xc >= q[0:1, :]) & (xc < q[2:3, :])
                  & (yc >= q[1:2, :]) & (yc < q[3:4, :]))

        key = jnp.where(in_ign, jnp.where(in_pos, rank, 1000.0), 1e9)
        sel = jnp.min(key, axis=1, keepdims=True)
        onehot = (rank == sel).astype(jnp.float32)
        sel_pos = (sel < 1000.0).astype(jnp.float32)

        coords = jnp.dot(onehot, box_cols[:, 0:4],
                         preferred_element_type=jnp.float32,
                         precision=jax.lax.Precision.HIGHEST)

        onehot_p = onehot * sel_pos
        cls_ref[0, k * TILE:(k + 1) * TILE, :] = jnp.dot(
            onehot_p, lmat[...], preferred_element_type=jnp.float32)

        ssxy = jnp.concatenate([meta[:, 2:4], meta[:, 2:4]], axis=1)
        sgn = jnp.where(
            jax.lax.broadcasted_iota(jnp.int32, (1, 4), 1) < 2, 1.0, -1.0)
        ltrb = (ssxy - coords) * sgn * (sel_pos * 0.25)

        cls_m = 1.0 - (sel == 1000.0).astype(jnp.float32)
        zero = jnp.zeros_like(sel_pos)
        aux_ref[0, k * TILE:(k + 1) * TILE, :] = jnp.concatenate(
            [ltrb, cls_m, sel_pos, zero, zero], axis=1)

        npos_parts.append(sel_pos)

    part = jnp.sum(sum(npos_parts)).reshape(1, 1, 1)

    @pl.when(c == 0)
    def _init():
        npos_ref[...] = part

    @pl.when(c != 0)
    def _acc():
        npos_ref[...] += part


def kernel(gt_box_levels, gt_boxes, feature_shapes):
    del feature_shapes
    batch, nb = gt_box_levels.shape
    rows = jnp.concatenate(
        [jnp.transpose(gt_boxes, (0, 2, 1)),
         gt_box_levels[:, None, :].astype(jnp.float32)],
        axis=1,
    )
    meta = jnp.asarray(_META)
    tile_lvl = jnp.asarray(_TILE_LVL)

    blk = GROUP * TILE
    grid_spec = pltpu.PrefetchScalarGridSpec(
        num_scalar_prefetch=1,
        grid=(batch, NUM_GROUPS),
        in_specs=[
            pl.BlockSpec((1, 6, nb), lambda b, c, s: (b, 0, 0)),
            pl.BlockSpec((1, nb, 5), lambda b, c, s: (b, 0, 0)),
            pl.BlockSpec((1, GROUP, TILE, 4),
                         lambda b, c, s: (c, 0, 0, 0)),
        ],
        out_specs=[
            pl.BlockSpec((1, blk, NUM_CLASSES), lambda b, c, s: (b, c, 0)),
            pl.BlockSpec((1, blk, 8), lambda b, c, s: (b, c, 0)),
            pl.BlockSpec((1, 1, 1), lambda b, c, s: (b, 0, 0)),
        ],
        scratch_shapes=[pltpu.VMEM((48, nb), jnp.float32),
                        pltpu.VMEM((nb, NUM_CLASSES), jnp.float32)],
    )

    cls_t, aux, num_pos = pl.pallas_call(
        _fsaf_group,
        grid_spec=grid_spec,
        out_shape=[
            jax.ShapeDtypeStruct((batch, NUM_LOC, NUM_CLASSES), jnp.float32),
            jax.ShapeDtypeStruct((batch, NUM_LOC, 8), jnp.float32),
            jax.ShapeDtypeStruct((batch, 1, 1), jnp.float32),
        ],
    )(tile_lvl, rows, gt_boxes, meta)

    return (cls_t,
            aux[..., 4] != 0.0,
            num_pos[:, 0, 0],
            aux[..., 0:4],
            aux[..., 5] != 0.0)


# speedup vs baseline: 1.4398x; 1.0792x over previous
import jax
import jax.numpy as jnp
import numpy as np
from jax.experimental import pallas as pl
from jax.experimental.pallas import tpu as pltpu

STRIDES = (8, 16, 32, 64, 128)
FEATURE_SHAPES = ((128, 128), (64, 64), (32, 32), (16, 16), (8, 8))
POS_SCALE = 0.2
IGNORE_SCALE = 0.5
NUM_CLASSES = 80
TILE = 64
NUM_LOC = sum(h * w for h, w in FEATURE_SHAPES)
NUM_TILES = NUM_LOC // TILE
GROUP = 12
NUM_GROUPS = -(-NUM_TILES // GROUP)


def _build_meta():
    cols = []
    for stride, (fh, fw) in zip(STRIDES, FEATURE_SHAPES):
        ys, xs = np.meshgrid(np.arange(fh), np.arange(fw), indexing="ij")
        xs = xs.reshape(-1).astype(np.float32)
        ys = ys.reshape(-1).astype(np.float32)
        cols.append(
            np.stack([xs, ys, (xs + 0.5) * stride, (ys + 0.5) * stride],
                     axis=1))
    meta = np.concatenate(cols, axis=0)
    pad = NUM_GROUPS * GROUP * TILE - NUM_LOC
    meta = np.concatenate([meta, np.full((pad, 4), -1.0, np.float32)],
                          axis=0)
    return meta.reshape(NUM_GROUPS, GROUP, TILE, 4)


def _build_tile_levels():
    lv = []
    for lid, (fh, fw) in enumerate(FEATURE_SHAPES):
        lv += [lid] * ((fh * fw) // TILE)
    lv += [len(STRIDES) - 1] * (NUM_GROUPS * GROUP - NUM_TILES)
    return np.asarray(lv, dtype=np.int32)


_META = _build_meta()
_TILE_LVL = _build_tile_levels()


def _fsaf_group(lvl_sm, box_rows_ref, box_cols_ref, meta_ref,
                cls_ref, aux_ref, npos_ref, bnd, lmat):
    c = pl.program_id(1)
    nb = box_cols_ref.shape[1]

    @pl.when(c == 0)
    def _prologue():
        rows = box_rows_ref[0]
        x1 = rows[0:1, :]
        y1 = rows[1:2, :]
        x2 = rows[2:3, :]
        y2 = rows[3:4, :]
        blv = rows[5:6, :]
        cols = box_cols_ref[0]
        a_col = ((cols[:, 2:3] - cols[:, 0:1])
                 * (cols[:, 3:4] - cols[:, 1:2]))
        a_row = (x2 - x1) * (y2 - y1)
        li = jax.lax.broadcasted_iota(jnp.int32, (nb, nb), 1)
        si = jax.lax.broadcasted_iota(jnp.int32, (nb, nb), 0)
        before = (a_col < a_row) | ((a_col == a_row) & (si < li))
        bnd[40:41, :] = jnp.sum(before.astype(jnp.float32), axis=0,
                                keepdims=True)
        cls_iota = jax.lax.broadcasted_iota(jnp.int32, (nb, NUM_CLASSES),
                                            1).astype(jnp.float32)
        lmat[...] = (cls_iota == cols[:, 4:5]).astype(jnp.float32)
        for lid, (stride, (fh, fw)) in enumerate(zip(STRIDES,
                                                     FEATURE_SHAPES)):
            inv = 1.0 / stride
            px1 = x1 * inv
            py1 = y1 * inv
            px2 = x2 * inv
            py2 = y2 * inv
            cx = (px1 + px2) * 0.5
            cy = (py1 + py2) * 0.5
            dw = px2 - px1
            dh = py2 - py1
            at_l = blv == float(lid)
            for scale, base in ((POS_SCALE, 0), (IGNORE_SCALE, 20)):
                hw = dw * scale * 0.5
                hh = dh * scale * 0.5
                bx1 = jnp.clip(jnp.floor(cx - hw), 0.0, fw - 1.0)
                by1 = jnp.clip(jnp.floor(cy - hh), 0.0, fh - 1.0)
                bx2 = jnp.clip(jnp.ceil(cx + hw), 1.0, float(fw))
                by2 = jnp.clip(jnp.ceil(cy + hh), 1.0, float(fh))
                bx2 = jnp.maximum(bx2, bx1 + 1.0)
                by2 = jnp.maximum(by2, by1 + 1.0)
                r = base + 4 * lid
                bnd[r:r + 1, :] = jnp.where(at_l, bx1, 1e9)
                bnd[r + 1:r + 2, :] = by1
                bnd[r + 2:r + 3, :] = bx2
                bnd[r + 3:r + 4, :] = by2

    box_cols = box_cols_ref[0]
    rank = bnd[40:41, :]
    npos_parts = []
    for k in range(GROUP):
        meta = meta_ref[0, k]
        xc = meta[:, 0:1]
        yc = meta[:, 1:2]
        sx = meta[:, 2:3]
        sy = meta[:, 3:4]

        lvl4 = lvl_sm[c * GROUP + k] * 4
        p = bnd[pl.ds(lvl4, 4), :]
        q = bnd[pl.ds(lvl4 + 20, 4), :]

        in_pos = ((xc >= p[0:1, :]) & (xc < p[2:3, :])
                  & (yc >= p[1:2, :]) & (yc < p[3:4, :]))
        in_ign = ((xc >= q[0:1, :]) & (xc < q[2:3, :])
                  & (yc >= q[1:2, :]) & (yc < q[3:4, :]))

        key = jnp.where(in_ign, jnp.where(in_pos, rank, 1000.0), 1e9)
        sel = jnp.min(key, axis=1, keepdims=True)
        onehot = (rank == sel).astype(jnp.float32)
        sel_pos = (sel < 1000.0).astype(jnp.float32)

        coords = jnp.dot(onehot, box_cols[:, 0:4],
                         preferred_element_type=jnp.float32,
                         precision=jax.lax.Precision.HIGHEST)

        onehot_p = onehot * sel_pos
        cls_ref[0, k * TILE:(k + 1) * TILE, :] = jnp.dot(
            onehot_p, lmat[...], preferred_element_type=jnp.float32)

        ssxy = jnp.concatenate([meta[:, 2:4], meta[:, 2:4]], axis=1)
        sgn = jnp.where(
            jax.lax.broadcasted_iota(jnp.int32, (1, 4), 1) < 2, 1.0, -1.0)
        ltrb = (ssxy - coords) * sgn * (sel_pos * 0.25)

        cls_m = 1.0 - (sel == 1000.0).astype(jnp.float32)
        zero = jnp.zeros_like(sel_pos)
        aux_ref[0, k * TILE:(k + 1) * TILE, :] = jnp.concatenate(
            [ltrb, cls_m, sel_pos, zero, zero], axis=1)

        npos_parts.append(sel_pos)

    part = jnp.sum(sum(npos_parts)).reshape(1, 1, 1)

    @pl.when(c == 0)
    def _init():
        npos_ref[...] = part

    @pl.when(c != 0)
    def _acc():
        npos_ref[...] += part


def kernel(gt_box_levels, gt_boxes, feature_shapes):
    del feature_shapes
    batch, nb = gt_box_levels.shape
    rows = jnp.concatenate(
        [jnp.transpose(gt_boxes, (0, 2, 1)),
         gt_box_levels[:, None, :].astype(jnp.float32)],
        axis=1,
    )
    meta = jnp.asarray(_META)
    tile_lvl = jnp.asarray(_TILE_LVL)

    blk = GROUP * TILE
    grid_spec = pltpu.PrefetchScalarGridSpec(
        num_scalar_prefetch=1,
        grid=(batch, NUM_GROUPS),
        in_specs=[
            pl.BlockSpec((1, 6, nb), lambda b, c, s: (b, 0, 0)),
            pl.BlockSpec((1, nb, 5), lambda b, c, s: (b, 0, 0)),
            pl.BlockSpec((1, GROUP, TILE, 4),
                         lambda b, c, s: (c, 0, 0, 0)),
        ],
        out_specs=[
            pl.BlockSpec((1, blk, NUM_CLASSES), lambda b, c, s: (b, c, 0)),
            pl.BlockSpec((1, blk, 8), lambda b, c, s: (b, c, 0)),
            pl.BlockSpec((1, 1, 1), lambda b, c, s: (b, 0, 0)),
        ],
        scratch_shapes=[pltpu.VMEM((48, nb), jnp.float32),
                        pltpu.VMEM((nb, NUM_CLASSES), jnp.float32)],
    )

    cls_t, aux, num_pos = pl.pallas_call(
        _fsaf_group,
        grid_spec=grid_spec,
        out_shape=[
            jax.ShapeDtypeStruct((batch, NUM_LOC, NUM_CLASSES), jnp.float32),
            jax.ShapeDtypeStruct((batch, NUM_LOC, 8), jnp.float32),
            jax.ShapeDtypeStruct((batch, 1, 1), jnp.float32),
        ],
    )(tile_lvl, rows, gt_boxes, meta)

    return (cls_t,
            aux[..., 4] != 0.0,
            num_pos[:, 0, 0],
            aux[..., 0:4],
            aux[..., 5] != 0.0)


# speedup vs baseline: 1.4808x; 1.0285x over previous
import jax
import jax.numpy as jnp
import numpy as np
from jax.experimental import pallas as pl
from jax.experimental.pallas import tpu as pltpu

STRIDES = (8, 16, 32, 64, 128)
FEATURE_SHAPES = ((128, 128), (64, 64), (32, 32), (16, 16), (8, 8))
POS_SCALE = 0.2
IGNORE_SCALE = 0.5
NUM_CLASSES = 80
TILE = 64
NUM_LOC = sum(h * w for h, w in FEATURE_SHAPES)
NUM_TILES = NUM_LOC // TILE
GROUP = 14
NUM_GROUPS = -(-NUM_TILES // GROUP)


def _build_meta():
    cols = []
    for stride, (fh, fw) in zip(STRIDES, FEATURE_SHAPES):
        ys, xs = np.meshgrid(np.arange(fh), np.arange(fw), indexing="ij")
        xs = xs.reshape(-1).astype(np.float32)
        ys = ys.reshape(-1).astype(np.float32)
        cols.append(
            np.stack([xs, ys, (xs + 0.5) * stride, (ys + 0.5) * stride],
                     axis=1))
    meta = np.concatenate(cols, axis=0)
    pad = NUM_GROUPS * GROUP * TILE - NUM_LOC
    meta = np.concatenate([meta, np.full((pad, 4), -1.0, np.float32)],
                          axis=0)
    return meta.reshape(NUM_GROUPS, GROUP, TILE, 4)


def _build_tile_levels():
    lv = []
    for lid, (fh, fw) in enumerate(FEATURE_SHAPES):
        lv += [lid] * ((fh * fw) // TILE)
    lv += [len(STRIDES) - 1] * (NUM_GROUPS * GROUP - NUM_TILES)
    return np.asarray(lv, dtype=np.int32)


_META = _build_meta()
_TILE_LVL = _build_tile_levels()


def _fsaf_group(lvl_sm, box_rows_ref, box_cols_ref, meta_ref,
                cls_ref, aux_ref, npos_ref, bnd, lmat):
    c = pl.program_id(1)
    nb = box_cols_ref.shape[1]

    @pl.when(c == 0)
    def _prologue():
        rows = box_rows_ref[0]
        x1 = rows[0:1, :]
        y1 = rows[1:2, :]
        x2 = rows[2:3, :]
        y2 = rows[3:4, :]
        blv = rows[5:6, :]
        cols = box_cols_ref[0]
        a_col = ((cols[:, 2:3] - cols[:, 0:1])
                 * (cols[:, 3:4] - cols[:, 1:2]))
        a_row = (x2 - x1) * (y2 - y1)
        li = jax.lax.broadcasted_iota(jnp.int32, (nb, nb), 1)
        si = jax.lax.broadcasted_iota(jnp.int32, (nb, nb), 0)
        before = (a_col < a_row) | ((a_col == a_row) & (si < li))
        bnd[40:41, :] = jnp.sum(before.astype(jnp.float32), axis=0,
                                keepdims=True)
        cls_iota = jax.lax.broadcasted_iota(jnp.int32, (nb, NUM_CLASSES),
                                            1).astype(jnp.float32)
        lmat[...] = (cls_iota == cols[:, 4:5]).astype(jnp.float32)
        for lid, (stride, (fh, fw)) in enumerate(zip(STRIDES,
                                                     FEATURE_SHAPES)):
            inv = 1.0 / stride
            px1 = x1 * inv
            py1 = y1 * inv
            px2 = x2 * inv
            py2 = y2 * inv
            cx = (px1 + px2) * 0.5
            cy = (py1 + py2) * 0.5
            dw = px2 - px1
            dh = py2 - py1
            at_l = blv == float(lid)
            for scale, base in ((POS_SCALE, 0), (IGNORE_SCALE, 20)):
                hw = dw * scale * 0.5
                hh = dh * scale * 0.5
                bx1 = jnp.clip(jnp.floor(cx - hw), 0.0, fw - 1.0)
                by1 = jnp.clip(jnp.floor(cy - hh), 0.0, fh - 1.0)
                bx2 = jnp.clip(jnp.ceil(cx + hw), 1.0, float(fw))
                by2 = jnp.clip(jnp.ceil(cy + hh), 1.0, float(fh))
                bx2 = jnp.maximum(bx2, bx1 + 1.0)
                by2 = jnp.maximum(by2, by1 + 1.0)
                r = base + 4 * lid
                bnd[r:r + 1, :] = jnp.where(at_l, bx1, 1e9)
                bnd[r + 1:r + 2, :] = by1
                bnd[r + 2:r + 3, :] = bx2
                bnd[r + 3:r + 4, :] = by2

    box_cols = box_cols_ref[0]
    rank = bnd[40:41, :]
    npos_parts = []
    for k in range(GROUP):
        meta = meta_ref[0, k]
        xc = meta[:, 0:1]
        yc = meta[:, 1:2]
        sx = meta[:, 2:3]
        sy = meta[:, 3:4]

        lvl4 = lvl_sm[c * GROUP + k] * 4
        p = bnd[pl.ds(lvl4, 4), :]
        q = bnd[pl.ds(lvl4 + 20, 4), :]

        in_pos = ((xc >= p[0:1, :]) & (xc < p[2:3, :])
                  & (yc >= p[1:2, :]) & (yc < p[3:4, :]))
        in_ign = ((xc >= q[0:1, :]) & (xc < q[2:3, :])
                  & (yc >= q[1:2, :]) & (yc < q[3:4, :]))

        key = jnp.where(in_ign, jnp.where(in_pos, rank, 1000.0), 1e9)
        sel = jnp.min(key, axis=1, keepdims=True)
        onehot = (rank == sel).astype(jnp.float32)
        sel_pos = (sel < 1000.0).astype(jnp.float32)

        coords = jnp.dot(onehot, box_cols[:, 0:4],
                         preferred_element_type=jnp.float32,
                         precision=jax.lax.Precision.HIGHEST)

        onehot_p = onehot * sel_pos
        cls_ref[0, k * TILE:(k + 1) * TILE, :] = jnp.dot(
            onehot_p, lmat[...], preferred_element_type=jnp.float32)

        ssxy = jnp.concatenate([meta[:, 2:4], meta[:, 2:4]], axis=1)
        sgn = jnp.where(
            jax.lax.broadcasted_iota(jnp.int32, (1, 4), 1) < 2, 1.0, -1.0)
        ltrb = (ssxy - coords) * sgn * (sel_pos * 0.25)

        cls_m = 1.0 - (sel == 1000.0).astype(jnp.float32)
        zero = jnp.zeros_like(sel_pos)
        aux_ref[0, k * TILE:(k + 1) * TILE, :] = jnp.concatenate(
            [ltrb, cls_m, sel_pos, zero, zero], axis=1)

        npos_parts.append(sel_pos)

    part = jnp.sum(sum(npos_parts)).reshape(1, 1, 1)

    @pl.when(c == 0)
    def _init():
        npos_ref[...] = part

    @pl.when(c != 0)
    def _acc():
        npos_ref[...] += part


def kernel(gt_box_levels, gt_boxes, feature_shapes):
    del feature_shapes
    batch, nb = gt_box_levels.shape
    rows = jnp.concatenate(
        [jnp.transpose(gt_boxes, (0, 2, 1)),
         gt_box_levels[:, None, :].astype(jnp.float32)],
        axis=1,
    )
    meta = jnp.asarray(_META)
    tile_lvl = jnp.asarray(_TILE_LVL)

    blk = GROUP * TILE
    grid_spec = pltpu.PrefetchScalarGridSpec(
        num_scalar_prefetch=1,
        grid=(batch, NUM_GROUPS),
        in_specs=[
            pl.BlockSpec((1, 6, nb), lambda b, c, s: (b, 0, 0)),
            pl.BlockSpec((1, nb, 5), lambda b, c, s: (b, 0, 0)),
            pl.BlockSpec((1, GROUP, TILE, 4),
                         lambda b, c, s: (c, 0, 0, 0)),
        ],
        out_specs=[
            pl.BlockSpec((1, blk, NUM_CLASSES), lambda b, c, s: (b, c, 0)),
            pl.BlockSpec((1, blk, 8), lambda b, c, s: (b, c, 0)),
            pl.BlockSpec((1, 1, 1), lambda b, c, s: (b, 0, 0)),
        ],
        scratch_shapes=[pltpu.VMEM((48, nb), jnp.float32),
                        pltpu.VMEM((nb, NUM_CLASSES), jnp.float32)],
    )

    cls_t, aux, num_pos = pl.pallas_call(
        _fsaf_group,
        grid_spec=grid_spec,
        out_shape=[
            jax.ShapeDtypeStruct((batch, NUM_LOC, NUM_CLASSES), jnp.float32),
            jax.ShapeDtypeStruct((batch, NUM_LOC, 8), jnp.float32),
            jax.ShapeDtypeStruct((batch, 1, 1), jnp.float32),
        ],
    )(tile_lvl, rows, gt_boxes, meta)

    return (cls_t,
            aux[..., 4] != 0.0,
            num_pos[:, 0, 0],
            aux[..., 0:4],
            aux[..., 5] != 0.0)
